# Initial kernel scaffold; baseline (speedup 1.0000x reference)
import jax
import jax.numpy as jnp
from jax.experimental import pallas as pl


def kernel(h, edge_index, r, norm, emb_e, w1, loop_w1, bias1, w2, loop_w2, bias2):
    raise NotImplementedError("write your pallas kernel here")


# speedup vs baseline: 7.4683x; 7.4683x over previous
import functools

import jax
import jax.numpy as jnp
from jax import lax
from jax.experimental import pallas as pl
from jax.experimental.pallas import tpu as pltpu
from jax.experimental.pallas import tpu_sc as plsc

N = 10000
E = 320000
D = 128
NR = 200
NB = 4
BLK = D // NB

T = 512
NT = E // T
S = 832

NC = 2
NS = 16
NW = NC * NS
CH = 80

XPAD = 10240


@functools.lru_cache(maxsize=None)
def _make_sc_gather(n_rows):
    per_w = n_rows // NW
    n_ch = per_w // CH
    mesh = plsc.VectorSubcoreMesh(
        core_axis_name="c", subcore_axis_name="s", num_cores=NC, num_subcores=NS)

    @functools.partial(
        pl.kernel,
        out_type=jax.ShapeDtypeStruct((n_rows, D), jnp.float32),
        mesh=mesh,
        scratch_types=[
            pltpu.VMEM((CH,), jnp.int32),
            pltpu.VMEM((CH, D), jnp.float32),
            pltpu.SemaphoreType.DMA,
        ],
    )
    def gather_k(table_hbm, idx_hbm, out_hbm, idx_v, rows_v, sem):
        wid = lax.axis_index("s") * NC + lax.axis_index("c")
        base = wid * per_w

        def body(ci, carry):
            off = base + ci * CH
            pltpu.sync_copy(idx_hbm.at[pl.ds(off, CH)], idx_v)
            pltpu.async_copy(table_hbm.at[idx_v], rows_v, sem).wait()
            pltpu.sync_copy(rows_v, out_hbm.at[pl.ds(off, CH)])
            return carry

        lax.fori_loop(0, n_ch, body, 0)

    return gather_k


NPAD = XPAD
ROWS_PER_TILE = NPAD // NS


@functools.lru_cache(maxsize=None)
def _make_sc_scatter():
    mesh = plsc.VectorSubcoreMesh(
        core_axis_name="c", subcore_axis_name="s", num_cores=NC, num_subcores=NS)

    @functools.partial(
        pl.kernel,
        out_type=jax.ShapeDtypeStruct((NC, NPAD, D), jnp.float32),
        mesh=mesh,
        scratch_types=[
            pltpu.VMEM((CH,), jnp.int32),
            pltpu.VMEM((CH, D), jnp.float32),
            pltpu.VMEM_SHARED((NPAD, D), jnp.float32),
        ],
    )
    def scatter_k(msg_hbm, dst_hbm, zeros_hbm, out_hbm, idx_v, rows_v, acc_sh):
        core = lax.axis_index("c")
        sid = lax.axis_index("s")
        row0 = sid * ROWS_PER_TILE
        pltpu.sync_copy(zeros_hbm, acc_sh.at[pl.ds(row0, ROWS_PER_TILE)])
        plsc.subcore_barrier()

        wid = core * NS + sid
        base = wid * (E // NW)

        def body(ci, carry):
            off = base + ci * CH
            pltpu.sync_copy(dst_hbm.at[pl.ds(off, CH)], idx_v)
            pltpu.sync_copy(msg_hbm.at[pl.ds(off, CH)], rows_v)
            pltpu.sync_copy(rows_v, acc_sh.at[idx_v], add=True)
            return carry

        lax.fori_loop(0, (E // NW) // CH, body, 0)
        plsc.subcore_barrier()
        pltpu.sync_copy(
            acc_sh.at[pl.ds(row0, ROWS_PER_TILE)],
            out_hbm.at[core, pl.ds(row0, ROWS_PER_TILE)],
        )

    return scatter_k


def _bmm_body(ti_ref, ri_ref, rs_ref, re_ref, g_ref, norm_ref, w_ref, o_ref):
    s = pl.program_id(0)
    prev = ti_ref[jnp.maximum(s - 1, 0)]
    first = jnp.logical_or(s == 0, ti_ref[s] != prev)

    @pl.when(first)
    def _():
        o_ref[...] = jnp.zeros_like(o_ref)

    base = ti_ref[s] * T
    rows = lax.broadcasted_iota(jnp.int32, (T, 1), 0) + base
    mask = jnp.logical_and(rows >= rs_ref[s], rows < re_ref[s])
    gm = jnp.where(mask, g_ref[...], 0.0)
    contrib = jnp.dot(gm, w_ref[0], preferred_element_type=jnp.float32)
    o_ref[...] += contrib * norm_ref[...]


def _tc_bmm(step_tile, step_rel, step_start, step_end, g, norm_s, wd):
    grid_spec = pltpu.PrefetchScalarGridSpec(
        num_scalar_prefetch=4,
        grid=(S,),
        in_specs=[
            pl.BlockSpec((T, D), lambda s, ti, ri, rs, re: (ti[s], 0)),
            pl.BlockSpec((T, 1), lambda s, ti, ri, rs, re: (ti[s], 0)),
            pl.BlockSpec((1, D, D), lambda s, ti, ri, rs, re: (ri[s], 0, 0)),
        ],
        out_specs=pl.BlockSpec((T, D), lambda s, ti, ri, rs, re: (ti[s], 0)),
    )
    return pl.pallas_call(
        _bmm_body,
        grid_spec=grid_spec,
        out_shape=jax.ShapeDtypeStruct((E, D), jnp.float32),
    )(step_tile, step_rel, step_start, step_end, g, norm_s, wd)


def _finish_body_relu(agg_ref, x_ref, w_ref, b_ref, o_ref):
    acc = agg_ref[0] + agg_ref[1] + b_ref[...]
    acc += jnp.dot(x_ref[...], w_ref[...], preferred_element_type=jnp.float32)
    o_ref[...] = jnp.maximum(acc, 0.0)


def _finish_body_lin(agg_ref, x_ref, w_ref, b_ref, o_ref):
    acc = agg_ref[0] + agg_ref[1] + b_ref[...]
    acc += jnp.dot(x_ref[...], w_ref[...], preferred_element_type=jnp.float32)
    o_ref[...] = acc


def _tc_finish(aggpair, x, loop_w, bias2d, relu):
    body = _finish_body_relu if relu else _finish_body_lin
    return pl.pallas_call(
        body,
        out_shape=jax.ShapeDtypeStruct((N, D), jnp.float32),
    )(aggpair, x, loop_w, bias2d)


def _dense_blockdiag(w):
    wd = jnp.zeros((NR, D, D), jnp.float32)
    for b in range(NB):
        sl = slice(b * BLK, (b + 1) * BLK)
        wd = wd.at[:, sl, sl].set(w[:, b])
    return wd


def kernel(h, edge_index, r, norm, emb_e, w1, loop_w1, bias1, w2, loop_w2, bias2):
    src, dst = edge_index[0], edge_index[1]
    perm = jnp.argsort(r)
    r_s = r[perm]
    dst_s = dst[perm]
    norm_s = norm[perm]
    src_s = src[perm]
    src1_s = h[src_s]

    offsets = jnp.searchsorted(r_s, jnp.arange(NR, dtype=jnp.int32)).astype(jnp.int32)

    tile_starts = (jnp.arange(NT, dtype=jnp.int32) * T)
    cuts = jnp.sort(jnp.concatenate([tile_starts, offsets]))
    cuts_p = jnp.concatenate(
        [cuts, jnp.full((S + 1 - cuts.shape[0],), E, jnp.int32)])
    step_start = cuts_p[:S]
    step_end = cuts_p[1:S + 1]
    step_tile = jnp.minimum(step_start // T, NT - 1).astype(jnp.int32)
    step_rel = jnp.clip(
        jnp.searchsorted(offsets, step_start, side="right") - 1, 0, NR - 1
    ).astype(jnp.int32)

    wd1 = _dense_blockdiag(w1)
    wd2 = _dense_blockdiag(w2)
    bias1_2d = bias1.reshape(1, D)
    bias2_2d = bias2.reshape(1, D)
    zeros_blk = jnp.zeros((ROWS_PER_TILE, D), jnp.float32)

    gather_edges = _make_sc_gather(E)
    gather_nodes = _make_sc_gather(XPAD)
    sc_scatter = _make_sc_scatter()

    h_pad = jnp.concatenate([h, jnp.zeros((XPAD - N,), jnp.int32)])
    x0 = gather_nodes(emb_e, h_pad)[:N]

    g1 = gather_edges(emb_e, src1_s)
    msg1 = _tc_bmm(step_tile, step_rel, step_start, step_end, g1, norm_s, wd1)
    aggp1 = sc_scatter(msg1, dst_s, zeros_blk)[:, :N]
    x1 = _tc_finish(aggp1, x0, loop_w1, bias1_2d, relu=True)

    g2 = gather_edges(x1, src_s)
    msg2 = _tc_bmm(step_tile, step_rel, step_start, step_end, g2, norm_s, wd2)
    aggp2 = sc_scatter(msg2, dst_s, zeros_blk)[:, :N]
    out = _tc_finish(aggp2, x1, loop_w2, bias2_2d, relu=False)
    return out


# speedup vs baseline: 11.4613x; 1.5347x over previous
import functools

import jax
import jax.numpy as jnp
from jax import lax
from jax.experimental import pallas as pl
from jax.experimental.pallas import tpu as pltpu
from jax.experimental.pallas import tpu_sc as plsc

N = 10000
E = 320000
D = 128
NR = 200
NB = 4
BLK = D // NB

T = 512
NT = E // T
S = 832

NC = 2
NS = 16
NW = NC * NS
CH = 80

XPAD = 10240


@functools.lru_cache(maxsize=None)
def _make_sc_gather(n_rows):
    per_w = n_rows // NW
    n_ch = per_w // CH
    mesh = plsc.VectorSubcoreMesh(
        core_axis_name="c", subcore_axis_name="s", num_cores=NC, num_subcores=NS)

    @functools.partial(
        pl.kernel,
        out_type=jax.ShapeDtypeStruct((n_rows, D), jnp.float32),
        mesh=mesh,
        scratch_types=[
            pltpu.VMEM((CH,), jnp.int32),
            pltpu.VMEM((CH, D), jnp.float32),
            pltpu.SemaphoreType.DMA,
        ],
    )
    def gather_k(table_hbm, idx_hbm, out_hbm, idx_v, rows_v, sem):
        wid = lax.axis_index("s") * NC + lax.axis_index("c")
        base = wid * per_w

        def body(ci, carry):
            off = base + ci * CH
            pltpu.sync_copy(idx_hbm.at[pl.ds(off, CH)], idx_v)
            pltpu.async_copy(table_hbm.at[idx_v], rows_v, sem).wait()
            pltpu.sync_copy(rows_v, out_hbm.at[pl.ds(off, CH)])
            return carry

        lax.fori_loop(0, n_ch, body, 0)

    return gather_k


@functools.lru_cache(maxsize=None)
def _make_sc_permute_attrs():
    per_w = E // NW
    n_ch = per_w // CH
    mesh = plsc.VectorSubcoreMesh(
        core_axis_name="c", subcore_axis_name="s", num_cores=NC, num_subcores=NS)

    @functools.partial(
        pl.kernel,
        out_type=(
            jax.ShapeDtypeStruct((E,), jnp.int32),
            jax.ShapeDtypeStruct((E,), jnp.int32),
            jax.ShapeDtypeStruct((E,), jnp.float32),
        ),
        mesh=mesh,
        scratch_types=[
            pltpu.VMEM((CH,), jnp.int32),
            pltpu.VMEM((CH,), jnp.int32),
            pltpu.VMEM((CH,), jnp.int32),
            pltpu.VMEM((CH,), jnp.float32),
            pltpu.SemaphoreType.DMA,
        ],
    )
    def permute_k(src_hbm, dst_hbm, nrm_hbm, perm_hbm,
                  osrc_hbm, odst_hbm, onrm_hbm,
                  idx_v, s_v, d_v, n_v, sem):
        wid = lax.axis_index("s") * NC + lax.axis_index("c")
        base = wid * per_w

        def body(ci, carry):
            off = base + ci * CH
            pltpu.sync_copy(perm_hbm.at[pl.ds(off, CH)], idx_v)
            pltpu.async_copy(src_hbm.at[idx_v], s_v, sem).wait()
            pltpu.async_copy(dst_hbm.at[idx_v], d_v, sem).wait()
            pltpu.async_copy(nrm_hbm.at[idx_v], n_v, sem).wait()
            pltpu.sync_copy(s_v, osrc_hbm.at[pl.ds(off, CH)])
            pltpu.sync_copy(d_v, odst_hbm.at[pl.ds(off, CH)])
            pltpu.sync_copy(n_v, onrm_hbm.at[pl.ds(off, CH)])
            return carry

        lax.fori_loop(0, n_ch, body, 0)

    return permute_k


NPAD = XPAD
ROWS_PER_TILE = NPAD // NS


@functools.lru_cache(maxsize=None)
def _make_sc_scatter():
    mesh = plsc.VectorSubcoreMesh(
        core_axis_name="c", subcore_axis_name="s", num_cores=NC, num_subcores=NS)

    @functools.partial(
        pl.kernel,
        out_type=jax.ShapeDtypeStruct((NC, NPAD, D), jnp.float32),
        mesh=mesh,
        scratch_types=[
            pltpu.VMEM((CH,), jnp.int32),
            pltpu.VMEM((CH, D), jnp.float32),
            pltpu.VMEM_SHARED((NPAD, D), jnp.float32),
        ],
    )
    def scatter_k(msg_hbm, dst_hbm, zeros_hbm, out_hbm, idx_v, rows_v, acc_sh):
        core = lax.axis_index("c")
        sid = lax.axis_index("s")
        row0 = sid * ROWS_PER_TILE
        pltpu.sync_copy(zeros_hbm, acc_sh.at[pl.ds(row0, ROWS_PER_TILE)])
        plsc.subcore_barrier()

        wid = core * NS + sid
        base = wid * (E // NW)

        def body(ci, carry):
            off = base + ci * CH
            pltpu.sync_copy(dst_hbm.at[pl.ds(off, CH)], idx_v)
            pltpu.sync_copy(msg_hbm.at[pl.ds(off, CH)], rows_v)
            pltpu.sync_copy(rows_v, acc_sh.at[idx_v], add=True)
            return carry

        lax.fori_loop(0, (E // NW) // CH, body, 0)
        plsc.subcore_barrier()
        pltpu.sync_copy(
            acc_sh.at[pl.ds(row0, ROWS_PER_TILE)],
            out_hbm.at[core, pl.ds(row0, ROWS_PER_TILE)],
        )

    return scatter_k


def _bmm_body(ti_ref, ri_ref, rs_ref, re_ref, g_ref, norm_ref, w_ref, o_ref):
    s = pl.program_id(0)
    prev = ti_ref[jnp.maximum(s - 1, 0)]
    first = jnp.logical_or(s == 0, ti_ref[s] != prev)

    @pl.when(first)
    def _():
        o_ref[...] = jnp.zeros_like(o_ref)

    base = ti_ref[s] * T
    rows = lax.broadcasted_iota(jnp.int32, (T, 1), 0) + base
    mask = jnp.logical_and(rows >= rs_ref[s], rows < re_ref[s])
    gm = jnp.where(mask, g_ref[...], 0.0)
    contrib = jnp.dot(gm, w_ref[0], preferred_element_type=jnp.float32)
    o_ref[...] += contrib * norm_ref[...]


def _tc_bmm(step_tile, step_rel, step_start, step_end, g, norm_s, wd):
    grid_spec = pltpu.PrefetchScalarGridSpec(
        num_scalar_prefetch=4,
        grid=(S,),
        in_specs=[
            pl.BlockSpec((T, D), lambda s, ti, ri, rs, re: (ti[s], 0)),
            pl.BlockSpec((T, 1), lambda s, ti, ri, rs, re: (ti[s], 0)),
            pl.BlockSpec((1, D, D), lambda s, ti, ri, rs, re: (ri[s], 0, 0)),
        ],
        out_specs=pl.BlockSpec((T, D), lambda s, ti, ri, rs, re: (ti[s], 0)),
    )
    return pl.pallas_call(
        _bmm_body,
        grid_spec=grid_spec,
        out_shape=jax.ShapeDtypeStruct((E, D), jnp.float32),
    )(step_tile, step_rel, step_start, step_end, g, norm_s, wd)


def _finish_body_relu(agg_ref, x_ref, w_ref, b_ref, o_ref):
    acc = agg_ref[0] + agg_ref[1] + b_ref[...]
    acc += jnp.dot(x_ref[...], w_ref[...], preferred_element_type=jnp.float32)
    o_ref[...] = jnp.maximum(acc, 0.0)


def _finish_body_lin(agg_ref, x_ref, w_ref, b_ref, o_ref):
    acc = agg_ref[0] + agg_ref[1] + b_ref[...]
    acc += jnp.dot(x_ref[...], w_ref[...], preferred_element_type=jnp.float32)
    o_ref[...] = acc


def _tc_finish(aggpair, x, loop_w, bias2d, relu):
    body = _finish_body_relu if relu else _finish_body_lin
    return pl.pallas_call(
        body,
        out_shape=jax.ShapeDtypeStruct((N, D), jnp.float32),
    )(aggpair, x, loop_w, bias2d)


def _dense_blockdiag(w):
    wd = jnp.zeros((NR, D, D), jnp.float32)
    for b in range(NB):
        sl = slice(b * BLK, (b + 1) * BLK)
        wd = wd.at[:, sl, sl].set(w[:, b])
    return wd


def kernel(h, edge_index, r, norm, emb_e, w1, loop_w1, bias1, w2, loop_w2, bias2):
    src, dst = edge_index[0], edge_index[1]
    r_s, perm = lax.sort_key_val(r, jnp.arange(E, dtype=jnp.int32))

    src_s, dst_s, norm1_s = _make_sc_permute_attrs()(src, dst, norm[:, 0], perm)
    norm_s = norm1_s.reshape(E, 1)

    offsets = jnp.searchsorted(r_s, jnp.arange(NR, dtype=jnp.int32)).astype(jnp.int32)

    tile_starts = (jnp.arange(NT, dtype=jnp.int32) * T)
    cuts = jnp.sort(jnp.concatenate([tile_starts, offsets]))
    cuts_p = jnp.concatenate(
        [cuts, jnp.full((S + 1 - cuts.shape[0],), E, jnp.int32)])
    step_start = cuts_p[:S]
    step_end = cuts_p[1:S + 1]
    step_tile = jnp.minimum(step_start // T, NT - 1).astype(jnp.int32)
    step_rel = jnp.clip(
        jnp.searchsorted(offsets, step_start, side="right") - 1, 0, NR - 1
    ).astype(jnp.int32)

    wd1 = _dense_blockdiag(w1)
    wd2 = _dense_blockdiag(w2)
    bias1_2d = bias1.reshape(1, D)
    bias2_2d = bias2.reshape(1, D)
    zeros_blk = jnp.zeros((ROWS_PER_TILE, D), jnp.float32)

    gather_edges = _make_sc_gather(E)
    gather_nodes = _make_sc_gather(XPAD)
    sc_scatter = _make_sc_scatter()

    h_pad = jnp.concatenate([h, jnp.zeros((XPAD - N,), jnp.int32)])
    x0p = gather_nodes(emb_e, h_pad)
    x0 = x0p[:N]

    g1 = gather_edges(x0p, src_s)
    msg1 = _tc_bmm(step_tile, step_rel, step_start, step_end, g1, norm_s, wd1)
    aggp1 = sc_scatter(msg1, dst_s, zeros_blk)[:, :N]
    x1 = _tc_finish(aggp1, x0, loop_w1, bias1_2d, relu=True)

    g2 = gather_edges(x1, src_s)
    msg2 = _tc_bmm(step_tile, step_rel, step_start, step_end, g2, norm_s, wd2)
    aggp2 = sc_scatter(msg2, dst_s, zeros_blk)[:, :N]
    out = _tc_finish(aggp2, x1, loop_w2, bias2_2d, relu=False)
    return out


# speedup vs baseline: 11.4698x; 1.0007x over previous
import functools

import jax
import jax.numpy as jnp
from jax import lax
from jax.experimental import pallas as pl
from jax.experimental.pallas import tpu as pltpu
from jax.experimental.pallas import tpu_sc as plsc

N = 10000
E = 320000
D = 128
NR = 200
NB = 4
BLK = D // NB

T = 512
NT = E // T
S = 832

NC = 2
NS = 16
NW = NC * NS
CH = 80

XPAD = 10240


@functools.lru_cache(maxsize=None)
def _make_sc_gather(n_rows):
    per_w = n_rows // NW
    n_ch = per_w // CH
    mesh = plsc.VectorSubcoreMesh(
        core_axis_name="c", subcore_axis_name="s", num_cores=NC, num_subcores=NS)

    @functools.partial(
        pl.kernel,
        out_type=jax.ShapeDtypeStruct((n_rows, D), jnp.float32),
        mesh=mesh,
        scratch_types=[
            pltpu.VMEM((CH,), jnp.int32),
            pltpu.VMEM((CH, D), jnp.float32),
            pltpu.SemaphoreType.DMA,
        ],
    )
    def gather_k(table_hbm, idx_hbm, out_hbm, idx_v, rows_v, sem):
        wid = lax.axis_index("s") * NC + lax.axis_index("c")
        base = wid * per_w

        def body(ci, carry):
            off = base + ci * CH
            pltpu.sync_copy(idx_hbm.at[pl.ds(off, CH)], idx_v)
            pltpu.async_copy(table_hbm.at[idx_v], rows_v, sem).wait()
            pltpu.sync_copy(rows_v, out_hbm.at[pl.ds(off, CH)])
            return carry

        lax.fori_loop(0, n_ch, body, 0)

    return gather_k


@functools.lru_cache(maxsize=None)
def _make_sc_permute_attrs():
    per_w = E // NW
    n_ch = per_w // CH
    mesh = plsc.VectorSubcoreMesh(
        core_axis_name="c", subcore_axis_name="s", num_cores=NC, num_subcores=NS)

    @functools.partial(
        pl.kernel,
        out_type=(
            jax.ShapeDtypeStruct((E,), jnp.int32),
            jax.ShapeDtypeStruct((E,), jnp.int32),
            jax.ShapeDtypeStruct((E,), jnp.float32),
        ),
        mesh=mesh,
        scratch_types=[
            pltpu.VMEM((CH,), jnp.int32),
            pltpu.VMEM((CH,), jnp.int32),
            pltpu.VMEM((CH,), jnp.int32),
            pltpu.VMEM((CH,), jnp.float32),
            pltpu.SemaphoreType.DMA,
        ],
    )
    def permute_k(src_hbm, dst_hbm, nrm_hbm, perm_hbm,
                  osrc_hbm, odst_hbm, onrm_hbm,
                  idx_v, s_v, d_v, n_v, sem):
        wid = lax.axis_index("s") * NC + lax.axis_index("c")
        base = wid * per_w

        def body(ci, carry):
            off = base + ci * CH
            pltpu.sync_copy(perm_hbm.at[pl.ds(off, CH)], idx_v)
            pltpu.async_copy(src_hbm.at[idx_v], s_v, sem).wait()
            pltpu.async_copy(dst_hbm.at[idx_v], d_v, sem).wait()
            pltpu.async_copy(nrm_hbm.at[idx_v], n_v, sem).wait()
            pltpu.sync_copy(s_v, osrc_hbm.at[pl.ds(off, CH)])
            pltpu.sync_copy(d_v, odst_hbm.at[pl.ds(off, CH)])
            pltpu.sync_copy(n_v, onrm_hbm.at[pl.ds(off, CH)])
            return carry

        lax.fori_loop(0, n_ch, body, 0)

    return permute_k


NPAD = XPAD
ROWS_PER_TILE = NPAD // NS


@functools.lru_cache(maxsize=None)
def _make_sc_scatter():
    mesh = plsc.VectorSubcoreMesh(
        core_axis_name="c", subcore_axis_name="s", num_cores=NC, num_subcores=NS)

    @functools.partial(
        pl.kernel,
        out_type=jax.ShapeDtypeStruct((NC, NPAD, D), jnp.float32),
        mesh=mesh,
        scratch_types=[
            pltpu.VMEM((CH,), jnp.int32),
            pltpu.VMEM((CH, D), jnp.float32),
            pltpu.VMEM_SHARED((NPAD, D), jnp.float32),
        ],
    )
    def scatter_k(msg_hbm, dst_hbm, zeros_hbm, out_hbm, idx_v, rows_v, acc_sh):
        core = lax.axis_index("c")
        sid = lax.axis_index("s")
        row0 = sid * ROWS_PER_TILE
        pltpu.sync_copy(zeros_hbm, acc_sh.at[pl.ds(row0, ROWS_PER_TILE)])
        plsc.subcore_barrier()

        wid = core * NS + sid
        base = wid * (E // NW)

        def body(ci, carry):
            off = base + ci * CH
            pltpu.sync_copy(dst_hbm.at[pl.ds(off, CH)], idx_v)
            pltpu.sync_copy(msg_hbm.at[pl.ds(off, CH)], rows_v)
            pltpu.sync_copy(rows_v, acc_sh.at[idx_v], add=True)
            return carry

        lax.fori_loop(0, (E // NW) // CH, body, 0)
        plsc.subcore_barrier()
        pltpu.sync_copy(
            acc_sh.at[pl.ds(row0, ROWS_PER_TILE)],
            out_hbm.at[core, pl.ds(row0, ROWS_PER_TILE)],
        )

    return scatter_k


def _bmm_body(ti_ref, ri_ref, rs_ref, re_ref, g_ref, norm_ref, w_ref, o_ref):
    s = pl.program_id(0)
    prev = ti_ref[jnp.maximum(s - 1, 0)]
    first = jnp.logical_or(s == 0, ti_ref[s] != prev)

    @pl.when(first)
    def _():
        o_ref[...] = jnp.zeros_like(o_ref)

    base = ti_ref[s] * T
    rows = lax.broadcasted_iota(jnp.int32, (T, 1), 0)
    mask = jnp.logical_and(rows >= rs_ref[s] - base, rows < re_ref[s] - base)
    nm = jnp.where(mask, norm_ref[...], 0.0)
    contrib = jnp.dot(g_ref[...], w_ref[0], preferred_element_type=jnp.float32)
    o_ref[...] += contrib * nm


def _tc_bmm(step_tile, step_rel, step_start, step_end, g, norm_s, wd):
    grid_spec = pltpu.PrefetchScalarGridSpec(
        num_scalar_prefetch=4,
        grid=(S,),
        in_specs=[
            pl.BlockSpec((T, D), lambda s, ti, ri, rs, re: (ti[s], 0)),
            pl.BlockSpec((T, 1), lambda s, ti, ri, rs, re: (ti[s], 0)),
            pl.BlockSpec((1, D, D), lambda s, ti, ri, rs, re: (ri[s], 0, 0)),
        ],
        out_specs=pl.BlockSpec((T, D), lambda s, ti, ri, rs, re: (ti[s], 0)),
    )
    return pl.pallas_call(
        _bmm_body,
        grid_spec=grid_spec,
        out_shape=jax.ShapeDtypeStruct((E, D), jnp.float32),
    )(step_tile, step_rel, step_start, step_end, g, norm_s, wd)


def _finish_body_relu(agg_ref, x_ref, w_ref, b_ref, o_ref):
    acc = agg_ref[0] + agg_ref[1] + b_ref[...]
    acc += jnp.dot(x_ref[...], w_ref[...], preferred_element_type=jnp.float32)
    o_ref[...] = jnp.maximum(acc, 0.0)


def _finish_body_lin(agg_ref, x_ref, w_ref, b_ref, o_ref):
    acc = agg_ref[0] + agg_ref[1] + b_ref[...]
    acc += jnp.dot(x_ref[...], w_ref[...], preferred_element_type=jnp.float32)
    o_ref[...] = acc


def _tc_finish(aggpair, x, loop_w, bias2d, relu):
    body = _finish_body_relu if relu else _finish_body_lin
    return pl.pallas_call(
        body,
        out_shape=jax.ShapeDtypeStruct((N, D), jnp.float32),
    )(aggpair, x, loop_w, bias2d)


def _dense_blockdiag(w):
    wd = jnp.zeros((NR, D, D), jnp.float32)
    for b in range(NB):
        sl = slice(b * BLK, (b + 1) * BLK)
        wd = wd.at[:, sl, sl].set(w[:, b])
    return wd


def kernel(h, edge_index, r, norm, emb_e, w1, loop_w1, bias1, w2, loop_w2, bias2):
    src, dst = edge_index[0], edge_index[1]
    r_s, perm = lax.sort_key_val(r, jnp.arange(E, dtype=jnp.int32))

    src_s, dst_s, norm1_s = _make_sc_permute_attrs()(src, dst, norm[:, 0], perm)
    norm_s = norm1_s.reshape(E, 1)

    offsets = jnp.searchsorted(r_s, jnp.arange(NR, dtype=jnp.int32)).astype(jnp.int32)

    tile_starts = (jnp.arange(NT, dtype=jnp.int32) * T)
    cuts = jnp.sort(jnp.concatenate([tile_starts, offsets]))
    cuts_p = jnp.concatenate(
        [cuts, jnp.full((S + 1 - cuts.shape[0],), E, jnp.int32)])
    step_start = cuts_p[:S]
    step_end = cuts_p[1:S + 1]
    step_tile = jnp.minimum(step_start // T, NT - 1).astype(jnp.int32)
    step_rel = jnp.clip(
        jnp.searchsorted(offsets, step_start, side="right") - 1, 0, NR - 1
    ).astype(jnp.int32)

    wd1 = _dense_blockdiag(w1)
    wd2 = _dense_blockdiag(w2)
    bias1_2d = bias1.reshape(1, D)
    bias2_2d = bias2.reshape(1, D)
    zeros_blk = jnp.zeros((ROWS_PER_TILE, D), jnp.float32)

    gather_edges = _make_sc_gather(E)
    gather_nodes = _make_sc_gather(XPAD)
    sc_scatter = _make_sc_scatter()

    h_pad = jnp.concatenate([h, jnp.zeros((XPAD - N,), jnp.int32)])
    x0p = gather_nodes(emb_e, h_pad)
    x0 = x0p[:N]

    g1 = gather_edges(x0p, src_s)
    msg1 = _tc_bmm(step_tile, step_rel, step_start, step_end, g1, norm_s, wd1)
    aggp1 = sc_scatter(msg1, dst_s, zeros_blk)[:, :N]
    x1 = _tc_finish(aggp1, x0, loop_w1, bias1_2d, relu=True)

    g2 = gather_edges(x1, src_s)
    msg2 = _tc_bmm(step_tile, step_rel, step_start, step_end, g2, norm_s, wd2)
    aggp2 = sc_scatter(msg2, dst_s, zeros_blk)[:, :N]
    out = _tc_finish(aggp2, x1, loop_w2, bias2_2d, relu=False)
    return out


# speedup vs baseline: 13.3483x; 1.1638x over previous
import functools

import jax
import jax.numpy as jnp
from jax import lax
from jax.experimental import pallas as pl
from jax.experimental.pallas import tpu as pltpu
from jax.experimental.pallas import tpu_sc as plsc

N = 10000
E = 320000
D = 128
NR = 200
NB = 4
BLK = D // NB

T = 512
NT = E // T
S = 832

NC = 2
NS = 16
NW = NC * NS
CH = 80

XPAD = 10240


@functools.lru_cache(maxsize=None)
def _make_sc_gather(n_rows):
    per_w = n_rows // NW
    n_ch = per_w // CH
    mesh = plsc.VectorSubcoreMesh(
        core_axis_name="c", subcore_axis_name="s", num_cores=NC, num_subcores=NS)

    @functools.partial(
        pl.kernel,
        out_type=jax.ShapeDtypeStruct((n_rows, D), jnp.float32),
        mesh=mesh,
        scratch_types=[
            pltpu.VMEM((n_ch, CH), jnp.int32),
            pltpu.VMEM((CH, D), jnp.float32),
            pltpu.VMEM((CH, D), jnp.float32),
            pltpu.SemaphoreType.DMA,
            pltpu.SemaphoreType.DMA,
            pltpu.SemaphoreType.DMA,
            pltpu.SemaphoreType.DMA,
        ],
    )
    def gather_k(table_hbm, idx_hbm, out_hbm, idx_v, rows0, rows1, sg0, sg1,
                 sw0, sw1):
        wid = lax.axis_index("s") * NC + lax.axis_index("c")
        base = wid * per_w
        pltpu.sync_copy(idx_hbm.at[wid], idx_v)

        def gath(ci, buf, sem):
            return pltpu.async_copy(table_hbm.at[idx_v.at[ci]], buf, sem)

        def wb(ci, buf, sem):
            return pltpu.async_copy(buf, out_hbm.at[pl.ds(base + ci * CH, CH)],
                                    sem)

        gath(0, rows0, sg0)

        def body(j, carry):
            ci0 = 2 * j
            ci1 = ci0 + 1

            @pl.when(j > 0)
            def _():
                pltpu.make_async_copy(rows1, out_hbm.at[pl.ds(base, CH)], sw1).wait()

            gath(ci1, rows1, sg1)
            pltpu.make_async_copy(table_hbm.at[idx_v.at[ci0]], rows0, sg0).wait()
            wb(ci0, rows0, sw0)
            pltpu.make_async_copy(rows0, out_hbm.at[pl.ds(base, CH)], sw0).wait()

            @pl.when(ci0 + 2 < n_ch)
            def _():
                gath(ci0 + 2, rows0, sg0)

            pltpu.make_async_copy(table_hbm.at[idx_v.at[ci1]], rows1, sg1).wait()
            wb(ci1, rows1, sw1)
            return carry

        lax.fori_loop(0, n_ch // 2, body, 0)
        pltpu.make_async_copy(rows1, out_hbm.at[pl.ds(base, CH)], sw1).wait()
        if n_ch % 2 == 1:
            ci = n_ch - 1
            pltpu.make_async_copy(table_hbm.at[idx_v.at[ci]], rows0, sg0).wait()
            pltpu.sync_copy(rows0, out_hbm.at[pl.ds(base + ci * CH, CH)])

    return gather_k


@functools.lru_cache(maxsize=None)
def _make_sc_permute_attrs():
    per_w = E // NW
    n_ch = per_w // CH
    mesh = plsc.VectorSubcoreMesh(
        core_axis_name="c", subcore_axis_name="s", num_cores=NC, num_subcores=NS)

    @functools.partial(
        pl.kernel,
        out_type=(
            jax.ShapeDtypeStruct((E,), jnp.int32),
            jax.ShapeDtypeStruct((E,), jnp.int32),
            jax.ShapeDtypeStruct((E,), jnp.float32),
        ),
        mesh=mesh,
        scratch_types=[
            pltpu.VMEM((n_ch, CH), jnp.int32),
            pltpu.VMEM((CH,), jnp.int32),
            pltpu.VMEM((CH,), jnp.int32),
            pltpu.VMEM((CH,), jnp.float32),
            pltpu.SemaphoreType.DMA,
            pltpu.SemaphoreType.DMA,
        ],
    )
    def permute_k(src_hbm, dst_hbm, nrm_hbm, perm_hbm,
                  osrc_hbm, odst_hbm, onrm_hbm,
                  idx_v, s_v, d_v, n_v, sg, sw):
        wid = lax.axis_index("s") * NC + lax.axis_index("c")
        base = wid * per_w
        pltpu.sync_copy(perm_hbm.at[wid], idx_v)

        def body(ci, carry):
            off = base + ci * CH

            @pl.when(ci > 0)
            def _():
                pltpu.make_async_copy(s_v, osrc_hbm.at[pl.ds(base, CH)], sw).wait()
                pltpu.make_async_copy(d_v, odst_hbm.at[pl.ds(base, CH)], sw).wait()
                pltpu.make_async_copy(n_v, onrm_hbm.at[pl.ds(base, CH)], sw).wait()

            cidx = idx_v.at[ci]
            pltpu.async_copy(src_hbm.at[cidx], s_v, sg)
            pltpu.async_copy(dst_hbm.at[cidx], d_v, sg)
            pltpu.async_copy(nrm_hbm.at[cidx], n_v, sg)
            pltpu.make_async_copy(src_hbm.at[cidx], s_v, sg).wait()
            pltpu.make_async_copy(dst_hbm.at[cidx], d_v, sg).wait()
            pltpu.make_async_copy(nrm_hbm.at[cidx], n_v, sg).wait()
            pltpu.async_copy(s_v, osrc_hbm.at[pl.ds(off, CH)], sw)
            pltpu.async_copy(d_v, odst_hbm.at[pl.ds(off, CH)], sw)
            pltpu.async_copy(n_v, onrm_hbm.at[pl.ds(off, CH)], sw)
            return carry

        lax.fori_loop(0, n_ch, body, 0)
        pltpu.make_async_copy(s_v, osrc_hbm.at[pl.ds(base, CH)], sw).wait()
        pltpu.make_async_copy(d_v, odst_hbm.at[pl.ds(base, CH)], sw).wait()
        pltpu.make_async_copy(n_v, onrm_hbm.at[pl.ds(base, CH)], sw).wait()

    return permute_k


NPAD = XPAD
ROWS_PER_TILE = NPAD // NS


@functools.lru_cache(maxsize=None)
def _make_sc_scatter():
    mesh = plsc.VectorSubcoreMesh(
        core_axis_name="c", subcore_axis_name="s", num_cores=NC, num_subcores=NS)

    @functools.partial(
        pl.kernel,
        out_type=jax.ShapeDtypeStruct((NC, NPAD, D), jnp.float32),
        mesh=mesh,
        scratch_types=[
            pltpu.VMEM((E // NW // CH, CH), jnp.int32),
            pltpu.VMEM((CH, D), jnp.float32),
            pltpu.VMEM((CH, D), jnp.float32),
            pltpu.VMEM_SHARED((NPAD, D), jnp.float32),
            pltpu.SemaphoreType.DMA,
            pltpu.SemaphoreType.DMA,
        ],
    )
    def scatter_k(msg_hbm, dst_hbm, zeros_hbm, out_hbm, idx_v, rows0, rows1,
                  acc_sh, sm0, sm1):
        core = lax.axis_index("c")
        sid = lax.axis_index("s")
        n_ch = E // NW // CH
        row0 = sid * ROWS_PER_TILE
        pltpu.sync_copy(zeros_hbm, acc_sh.at[pl.ds(row0, ROWS_PER_TILE)])

        wid = core * NS + sid
        base = wid * (E // NW)
        pltpu.sync_copy(dst_hbm.at[wid], idx_v)
        plsc.subcore_barrier()

        def load(ci, buf, sem):
            return pltpu.async_copy(msg_hbm.at[pl.ds(base + ci * CH, CH)], buf,
                                    sem)

        load(0, rows0, sm0)

        def body(j, carry):
            ci0 = 2 * j
            ci1 = ci0 + 1
            load(ci1, rows1, sm1)
            pltpu.make_async_copy(msg_hbm.at[pl.ds(base, CH)], rows0, sm0).wait()
            pltpu.sync_copy(rows0, acc_sh.at[idx_v.at[ci0]], add=True)

            @pl.when(ci0 + 2 < n_ch)
            def _():
                load(ci0 + 2, rows0, sm0)

            pltpu.make_async_copy(msg_hbm.at[pl.ds(base, CH)], rows1, sm1).wait()
            pltpu.sync_copy(rows1, acc_sh.at[idx_v.at[ci1]], add=True)
            return carry

        lax.fori_loop(0, n_ch // 2, body, 0)
        if n_ch % 2 == 1:
            ci = n_ch - 1
            pltpu.make_async_copy(msg_hbm.at[pl.ds(base, CH)], rows0, sm0).wait()
            pltpu.sync_copy(rows0, acc_sh.at[idx_v.at[ci]], add=True)
        plsc.subcore_barrier()
        pltpu.sync_copy(
            acc_sh.at[pl.ds(row0, ROWS_PER_TILE)],
            out_hbm.at[core, pl.ds(row0, ROWS_PER_TILE)],
        )

    return scatter_k


def _bmm_body(ti_ref, ri_ref, rs_ref, re_ref, g_ref, norm_ref, w_ref, o_ref):
    s = pl.program_id(0)
    prev = ti_ref[jnp.maximum(s - 1, 0)]
    first = jnp.logical_or(s == 0, ti_ref[s] != prev)

    @pl.when(first)
    def _():
        o_ref[...] = jnp.zeros_like(o_ref)

    base = ti_ref[s] * T
    rows = lax.broadcasted_iota(jnp.int32, (T, 1), 0)
    mask = jnp.logical_and(rows >= rs_ref[s] - base, rows < re_ref[s] - base)
    nm = jnp.where(mask, norm_ref[...], 0.0)
    contrib = jnp.dot(g_ref[...], w_ref[0], preferred_element_type=jnp.float32)
    o_ref[...] += contrib * nm


def _tc_bmm(step_tile, step_rel, step_start, step_end, g, norm_s, wd):
    grid_spec = pltpu.PrefetchScalarGridSpec(
        num_scalar_prefetch=4,
        grid=(S,),
        in_specs=[
            pl.BlockSpec((T, D), lambda s, ti, ri, rs, re: (ti[s], 0)),
            pl.BlockSpec((T, 1), lambda s, ti, ri, rs, re: (ti[s], 0)),
            pl.BlockSpec((1, D, D), lambda s, ti, ri, rs, re: (ri[s], 0, 0)),
        ],
        out_specs=pl.BlockSpec((T, D), lambda s, ti, ri, rs, re: (ti[s], 0)),
    )
    return pl.pallas_call(
        _bmm_body,
        grid_spec=grid_spec,
        out_shape=jax.ShapeDtypeStruct((E, D), jnp.float32),
    )(step_tile, step_rel, step_start, step_end, g, norm_s, wd)


def _finish_body_relu(agg_ref, x_ref, w_ref, b_ref, o_ref):
    acc = agg_ref[0] + agg_ref[1] + b_ref[...]
    acc += jnp.dot(x_ref[...], w_ref[...], preferred_element_type=jnp.float32)
    o_ref[...] = jnp.maximum(acc, 0.0)


def _finish_body_lin(agg_ref, x_ref, w_ref, b_ref, o_ref):
    acc = agg_ref[0] + agg_ref[1] + b_ref[...]
    acc += jnp.dot(x_ref[...], w_ref[...], preferred_element_type=jnp.float32)
    o_ref[...] = acc


def _tc_finish(aggpair, x, loop_w, bias2d, relu):
    body = _finish_body_relu if relu else _finish_body_lin
    return pl.pallas_call(
        body,
        out_shape=jax.ShapeDtypeStruct((N, D), jnp.float32),
    )(aggpair, x, loop_w, bias2d)


def _dense_blockdiag(w):
    wd = jnp.zeros((NR, D, D), jnp.float32)
    for b in range(NB):
        sl = slice(b * BLK, (b + 1) * BLK)
        wd = wd.at[:, sl, sl].set(w[:, b])
    return wd


def kernel(h, edge_index, r, norm, emb_e, w1, loop_w1, bias1, w2, loop_w2, bias2):
    src, dst = edge_index[0], edge_index[1]
    r_s, perm = lax.sort_key_val(r, jnp.arange(E, dtype=jnp.int32))

    perm3 = perm.reshape(NW, E // NW // CH, CH)
    src_s, dst_s, norm1_s = _make_sc_permute_attrs()(src, dst, norm[:, 0], perm3)
    norm_s = norm1_s.reshape(E, 1)
    src_s3 = src_s.reshape(NW, E // NW // CH, CH)
    dst_s3 = dst_s.reshape(NW, E // NW // CH, CH)

    offsets = jnp.searchsorted(r_s, jnp.arange(NR, dtype=jnp.int32)).astype(jnp.int32)

    tile_starts = (jnp.arange(NT, dtype=jnp.int32) * T)
    cuts = jnp.sort(jnp.concatenate([tile_starts, offsets]))
    cuts_p = jnp.concatenate(
        [cuts, jnp.full((S + 1 - cuts.shape[0],), E, jnp.int32)])
    step_start = cuts_p[:S]
    step_end = cuts_p[1:S + 1]
    step_tile = jnp.minimum(step_start // T, NT - 1).astype(jnp.int32)
    step_rel = jnp.clip(
        jnp.searchsorted(offsets, step_start, side="right") - 1, 0, NR - 1
    ).astype(jnp.int32)

    wd1 = _dense_blockdiag(w1)
    wd2 = _dense_blockdiag(w2)
    bias1_2d = bias1.reshape(1, D)
    bias2_2d = bias2.reshape(1, D)
    zeros_blk = jnp.zeros((ROWS_PER_TILE, D), jnp.float32)

    gather_edges = _make_sc_gather(E)
    gather_nodes = _make_sc_gather(XPAD)
    sc_scatter = _make_sc_scatter()

    h_pad = jnp.concatenate([h, jnp.zeros((XPAD - N,), jnp.int32)])
    h_pad3 = h_pad.reshape(NW, XPAD // NW // CH, CH)
    x0p = gather_nodes(emb_e, h_pad3)
    x0 = x0p[:N]

    g1 = gather_edges(x0p, src_s3)
    msg1 = _tc_bmm(step_tile, step_rel, step_start, step_end, g1, norm_s, wd1)
    aggp1 = sc_scatter(msg1, dst_s3, zeros_blk)[:, :N]
    x1 = _tc_finish(aggp1, x0, loop_w1, bias1_2d, relu=True)

    g2 = gather_edges(x1, src_s3)
    msg2 = _tc_bmm(step_tile, step_rel, step_start, step_end, g2, norm_s, wd2)
    aggp2 = sc_scatter(msg2, dst_s3, zeros_blk)[:, :N]
    out = _tc_finish(aggp2, x1, loop_w2, bias2_2d, relu=False)
    return out
